# initial kernel scaffold (unmeasured)
import functools

import jax
import jax.numpy as jnp
from jax import lax
from jax.experimental import pallas as pl
from jax.experimental.pallas import tpu as pltpu

N_DEV = 16
B, SQ, SKV, D_MODEL = 2, 128, 128, 512
HL, DH = 4, 64
ROWS = B * SQ
CH = ROWS // N_DEV


def kernel(x, Wq, K_ext, V_ext, Wo):
    my = lax.axis_index("i")
    K_loc = jnp.transpose(
        lax.dynamic_slice_in_dim(K_ext, my * HL, HL, axis=2), (0, 2, 1, 3)
    )
    V_loc = jnp.transpose(
        lax.dynamic_slice_in_dim(V_ext, my * HL, HL, axis=2), (0, 2, 1, 3)
    )
    x2d = x.reshape(ROWS, D_MODEL)

    def body(x_ref, wq_ref, k_ref, v_ref, wo_ref, out_ref,
             ctx_ref, partial_ref, rs_ref, acc_ref,
             send1, recv1, send2, recv2):
        my_pos = lax.axis_index("i")

        q_all = jnp.dot(x_ref[...], wq_ref[...],
                        preferred_element_type=jnp.float32)

        qi = lax.broadcasted_iota(jnp.int32, (SQ, SKV), 0)
        kj = lax.broadcasted_iota(jnp.int32, (SQ, SKV), 1)
        qb, kb = qi // 64, kj // 64
        mask = (qb == kb) | ((kb % 4) == (qb % 4))

        for b in range(B):
            for h in range(HL):
                q = q_all[b * SQ:(b + 1) * SQ, h * DH:(h + 1) * DH]
                k = k_ref[b, h]
                s = lax.dot_general(
                    q, k, (((1,), (1,)), ((), ())),
                    preferred_element_type=jnp.float32,
                ) * 0.125
                s = jnp.where(mask, s, -1e9)
                w = jnp.exp(s - jnp.max(s, axis=-1, keepdims=True))
                w = w / jnp.sum(w, axis=-1, keepdims=True)
                ctx = jnp.dot(w, v_ref[b, h],
                              preferred_element_type=jnp.float32)
                ctx_ref[b * SQ:(b + 1) * SQ, h * DH:(h + 1) * DH] = ctx

        partial_ref[...] = jnp.dot(ctx_ref[...], wo_ref[...],
                                   preferred_element_type=jnp.float32)

        for dst in range(N_DEV):
            @pl.when(dst != my_pos)
            def _():
                rdma = pltpu.make_async_remote_copy(
                    src_ref=partial_ref.at[pl.ds(dst * CH, CH), :],
                    dst_ref=rs_ref.at[my_pos],
                    send_sem=send1.at[dst],
                    recv_sem=recv1.at[my_pos],
                    device_id=(dst,),
                    device_id_type=pl.DeviceIdType.MESH,
                )
                rdma.start()

        rs_ref[pl.ds(my_pos, 1)] = partial_ref[
            pl.ds(my_pos * CH, CH), :
        ].reshape(1, CH, D_MODEL)
        for src in range(N_DEV):
            @pl.when(src != my_pos)
            def _():
                pltpu.make_async_remote_copy(
                    src_ref=partial_ref.at[pl.ds(0, CH), :],
                    dst_ref=rs_ref.at[src],
                    send_sem=send1.at[src],
                    recv_sem=recv1.at[src],
                    device_id=(src,),
                    device_id_type=pl.DeviceIdType.MESH,
                ).wait_recv()

        acc_ref[...] = jnp.sum(rs_ref[...], axis=0)

        for dst in range(N_DEV):
            @pl.when(dst != my_pos)
            def _():
                rdma = pltpu.make_async_remote_copy(
                    src_ref=acc_ref,
                    dst_ref=out_ref.at[pl.ds(my_pos * CH, CH), :],
                    send_sem=send2.at[dst],
                    recv_sem=recv2.at[my_pos],
                    device_id=(dst,),
                    device_id_type=pl.DeviceIdType.MESH,
                )
                rdma.start()

        out_ref[pl.ds(my_pos * CH, CH), :] = acc_ref[...]
        for src in range(N_DEV):
            @pl.when(src != my_pos)
            def _():
                pltpu.make_async_remote_copy(
                    src_ref=acc_ref,
                    dst_ref=out_ref.at[pl.ds(src * CH, CH), :],
                    send_sem=send2.at[src],
                    recv_sem=recv2.at[src],
                    device_id=(src,),
                    device_id_type=pl.DeviceIdType.MESH,
                ).wait_recv()

        for dst in range(N_DEV):
            @pl.when(dst != my_pos)
            def _():
                pltpu.make_async_remote_copy(
                    src_ref=partial_ref.at[pl.ds(dst * CH, CH), :],
                    dst_ref=rs_ref.at[my_pos],
                    send_sem=send1.at[dst],
                    recv_sem=recv1.at[my_pos],
                    device_id=(dst,),
                    device_id_type=pl.DeviceIdType.MESH,
                ).wait_send()
                pltpu.make_async_remote_copy(
                    src_ref=acc_ref,
                    dst_ref=out_ref.at[pl.ds(my_pos * CH, CH), :],
                    send_sem=send2.at[dst],
                    recv_sem=recv2.at[my_pos],
                    device_id=(dst,),
                    device_id_type=pl.DeviceIdType.MESH,
                ).wait_send()

    out = pl.pallas_call(
        body,
        out_shape=jax.ShapeDtypeStruct((ROWS, D_MODEL), jnp.float32),
        in_specs=[pl.BlockSpec(memory_space=pltpu.VMEM)] * 5,
        out_specs=pl.BlockSpec(memory_space=pltpu.VMEM),
        scratch_shapes=[
            pltpu.VMEM((ROWS, HL * DH), jnp.float32),
            pltpu.VMEM((ROWS, D_MODEL), jnp.float32),
            pltpu.VMEM((N_DEV, CH, D_MODEL), jnp.float32),
            pltpu.VMEM((CH, D_MODEL), jnp.float32),
            pltpu.SemaphoreType.DMA((N_DEV,)),
            pltpu.SemaphoreType.DMA((N_DEV,)),
            pltpu.SemaphoreType.DMA((N_DEV,)),
            pltpu.SemaphoreType.DMA((N_DEV,)),
        ],
        compiler_params=pltpu.CompilerParams(collective_id=0),
    )(x2d, Wq, K_loc, V_loc, Wo)
    return out.reshape(B, SQ, D_MODEL)


# baseline (device time: 31387 ns/iter reference)
import functools

import jax
import jax.numpy as jnp
from jax import lax
from jax.experimental import pallas as pl
from jax.experimental.pallas import tpu as pltpu

N_DEV = 16
B, SQ, SKV, D_MODEL = 2, 128, 128, 512
HL, DH = 4, 64
ROWS = B * SQ
CH = ROWS // N_DEV


def kernel(x, Wq, K_ext, V_ext, Wo):
    my = lax.axis_index("i")
    K_loc = jnp.transpose(
        lax.dynamic_slice_in_dim(K_ext, my * HL, HL, axis=2), (0, 2, 1, 3)
    )
    V_loc = jnp.transpose(
        lax.dynamic_slice_in_dim(V_ext, my * HL, HL, axis=2), (0, 2, 1, 3)
    )
    x2d = x.reshape(ROWS, D_MODEL)

    def body(x_ref, wq_ref, k_ref, v_ref, wo_ref, out_ref,
             ctx_ref, partial_ref, rs_ref, acc_ref,
             send1, recv1, send2, recv2):
        my_pos = lax.axis_index("i")

        q_all = jnp.dot(x_ref[...], wq_ref[...],
                        preferred_element_type=jnp.float32)

        qi = lax.broadcasted_iota(jnp.int32, (SQ, SKV), 0)
        kj = lax.broadcasted_iota(jnp.int32, (SQ, SKV), 1)
        qb, kb = qi // 64, kj // 64
        mask = (qb == kb) | ((kb % 4) == (qb % 4))

        for b in range(B):
            for h in range(HL):
                q = q_all[b * SQ:(b + 1) * SQ, h * DH:(h + 1) * DH]
                k = k_ref[b, h]
                s = lax.dot_general(
                    q, k, (((1,), (1,)), ((), ())),
                    preferred_element_type=jnp.float32,
                ) * 0.125
                s = jnp.where(mask, s, -1e9)
                w = jnp.exp(s - jnp.max(s, axis=-1, keepdims=True))
                w = w / jnp.sum(w, axis=-1, keepdims=True)
                ctx = jnp.dot(w, v_ref[b, h],
                              preferred_element_type=jnp.float32)
                ctx_ref[b * SQ:(b + 1) * SQ, h * DH:(h + 1) * DH] = ctx

        partial_ref[...] = jnp.dot(ctx_ref[...], wo_ref[...],
                                   preferred_element_type=jnp.float32)

        for dst in range(N_DEV):
            @pl.when(dst != my_pos)
            def _():
                rdma = pltpu.make_async_remote_copy(
                    src_ref=partial_ref.at[pl.ds(dst * CH, CH), :],
                    dst_ref=rs_ref.at[my_pos],
                    send_sem=send1.at[dst],
                    recv_sem=recv1.at[my_pos],
                    device_id=(dst,),
                    device_id_type=pl.DeviceIdType.MESH,
                )
                rdma.start()

        rs_ref[pl.ds(my_pos, 1)] = partial_ref[
            pl.ds(my_pos * CH, CH), :
        ].reshape(1, CH, D_MODEL)
        for src in range(N_DEV):
            @pl.when(src != my_pos)
            def _():
                pltpu.make_async_remote_copy(
                    src_ref=partial_ref.at[pl.ds(0, CH), :],
                    dst_ref=rs_ref.at[src],
                    send_sem=send1.at[src],
                    recv_sem=recv1.at[src],
                    device_id=(src,),
                    device_id_type=pl.DeviceIdType.MESH,
                ).wait_recv()

        acc_ref[...] = jnp.sum(rs_ref[...], axis=0)

        for dst in range(N_DEV):
            @pl.when(dst != my_pos)
            def _():
                rdma = pltpu.make_async_remote_copy(
                    src_ref=acc_ref,
                    dst_ref=out_ref.at[pl.ds(my_pos * CH, CH), :],
                    send_sem=send2.at[dst],
                    recv_sem=recv2.at[my_pos],
                    device_id=(dst,),
                    device_id_type=pl.DeviceIdType.MESH,
                )
                rdma.start()

        out_ref[pl.ds(my_pos * CH, CH), :] = acc_ref[...]
        for src in range(N_DEV):
            @pl.when(src != my_pos)
            def _():
                pltpu.make_async_remote_copy(
                    src_ref=acc_ref,
                    dst_ref=out_ref.at[pl.ds(src * CH, CH), :],
                    send_sem=send2.at[src],
                    recv_sem=recv2.at[src],
                    device_id=(src,),
                    device_id_type=pl.DeviceIdType.MESH,
                ).wait_recv()

        for dst in range(N_DEV):
            @pl.when(dst != my_pos)
            def _():
                pltpu.make_async_remote_copy(
                    src_ref=partial_ref.at[pl.ds(dst * CH, CH), :],
                    dst_ref=rs_ref.at[my_pos],
                    send_sem=send1.at[dst],
                    recv_sem=recv1.at[my_pos],
                    device_id=(dst,),
                    device_id_type=pl.DeviceIdType.MESH,
                ).wait_send()
                pltpu.make_async_remote_copy(
                    src_ref=acc_ref,
                    dst_ref=out_ref.at[pl.ds(my_pos * CH, CH), :],
                    send_sem=send2.at[dst],
                    recv_sem=recv2.at[my_pos],
                    device_id=(dst,),
                    device_id_type=pl.DeviceIdType.MESH,
                ).wait_send()

    out = pl.pallas_call(
        body,
        out_shape=jax.ShapeDtypeStruct((ROWS, D_MODEL), jnp.float32),
        in_specs=[pl.BlockSpec(memory_space=pltpu.VMEM)] * 5,
        out_specs=pl.BlockSpec(memory_space=pltpu.VMEM),
        scratch_shapes=[
            pltpu.VMEM((ROWS, HL * DH), jnp.float32),
            pltpu.VMEM((ROWS, D_MODEL), jnp.float32),
            pltpu.VMEM((N_DEV, CH, D_MODEL), jnp.float32),
            pltpu.VMEM((CH, D_MODEL), jnp.float32),
            pltpu.SemaphoreType.DMA((N_DEV,)),
            pltpu.SemaphoreType.DMA((N_DEV,)),
            pltpu.SemaphoreType.DMA((N_DEV,)),
            pltpu.SemaphoreType.DMA((N_DEV,)),
        ],
    )(x2d, Wq, K_loc, V_loc, Wo)
    return out.reshape(B, SQ, D_MODEL)


# device time: 20995 ns/iter; 1.4950x vs baseline; 1.4950x over previous
import jax
import jax.numpy as jnp
from jax import lax
from jax.experimental import pallas as pl
from jax.experimental.pallas import tpu as pltpu

N_DEV = 16
B, SQ, SKV, D_MODEL = 2, 128, 128, 512
HL, DH = 4, 64
ROWS = B * SQ
CH = ROWS // N_DEV
GRP = 4
DST_PER_GRP = N_DEV // GRP


def kernel(x, Wq, K_ext, V_ext, Wo):
    my = lax.axis_index("i")
    K_loc = lax.dynamic_slice_in_dim(K_ext, my * HL, HL, axis=2)
    V_loc = lax.dynamic_slice_in_dim(V_ext, my * HL, HL, axis=2)
    x2d = x.reshape(ROWS, D_MODEL)

    def body(x_ref, wq_ref, k_ref, v_ref, wo_ref, out_ref,
             ctx_ref, partial_ref, rs_ref, acc_ref, ag_ref,
             send1, recv1, send2, recv2):
        my_pos = lax.axis_index("i")

        barrier_sem = pltpu.get_barrier_semaphore()
        for peer in range(N_DEV):
            @pl.when(peer != my_pos)
            def _():
                pl.semaphore_signal(
                    barrier_sem, inc=1,
                    device_id=(peer,),
                    device_id_type=pl.DeviceIdType.MESH,
                )

        qi = lax.broadcasted_iota(jnp.int32, (SQ, SKV), 0)
        kj = lax.broadcasted_iota(jnp.int32, (SQ, SKV), 1)
        qb, kb = qi // 64, kj // 64
        mask = (qb == kb) | ((kb % 4) == (qb % 4))

        q_all = jnp.dot(x_ref[...], wq_ref[...],
                        preferred_element_type=jnp.float32)

        for b in range(B):
            for h in range(HL):
                q = q_all[b * SQ:(b + 1) * SQ, h * DH:(h + 1) * DH]
                k = k_ref[b, :, h, :]
                s = lax.dot_general(
                    q, k, (((1,), (1,)), ((), ())),
                    preferred_element_type=jnp.float32,
                ) * 0.125
                s = jnp.where(mask, s, -1e9)
                w = jnp.exp(s - jnp.max(s, axis=-1, keepdims=True))
                w = w / jnp.sum(w, axis=-1, keepdims=True)
                ctx = jnp.dot(w, v_ref[b, :, h, :],
                              preferred_element_type=jnp.float32)
                ctx_ref[b * SQ:(b + 1) * SQ, h * DH:(h + 1) * DH] = ctx

        pl.semaphore_wait(barrier_sem, N_DEV - 1)

        rpb = ROWS // GRP
        for g in range(GRP):
            partial_ref[g * rpb:(g + 1) * rpb, :] = jnp.dot(
                ctx_ref[g * rpb:(g + 1) * rpb, :], wo_ref[...],
                preferred_element_type=jnp.float32,
            ).astype(jnp.bfloat16)
            for dst in range(g * DST_PER_GRP, (g + 1) * DST_PER_GRP):
                @pl.when(dst != my_pos)
                def _():
                    pltpu.make_async_remote_copy(
                        src_ref=partial_ref.at[pl.ds(dst * CH, CH), :],
                        dst_ref=rs_ref.at[my_pos],
                        send_sem=send1.at[dst],
                        recv_sem=recv1.at[my_pos],
                        device_id=(dst,),
                        device_id_type=pl.DeviceIdType.MESH,
                    ).start()

        rs_ref[pl.ds(my_pos, 1)] = partial_ref[
            pl.ds(my_pos * CH, CH), :
        ].reshape(1, CH, D_MODEL)
        for src in range(N_DEV):
            @pl.when(src != my_pos)
            def _():
                pltpu.make_async_remote_copy(
                    src_ref=partial_ref.at[pl.ds(0, CH), :],
                    dst_ref=rs_ref.at[src],
                    send_sem=send1.at[src],
                    recv_sem=recv1.at[src],
                    device_id=(src,),
                    device_id_type=pl.DeviceIdType.MESH,
                ).wait_recv()

        acc_ref[...] = jnp.sum(
            rs_ref[...].astype(jnp.float32), axis=0
        ).astype(jnp.bfloat16)

        for dst in range(N_DEV):
            @pl.when(dst != my_pos)
            def _():
                pltpu.make_async_remote_copy(
                    src_ref=acc_ref,
                    dst_ref=ag_ref.at[my_pos],
                    send_sem=send2.at[dst],
                    recv_sem=recv2.at[my_pos],
                    device_id=(dst,),
                    device_id_type=pl.DeviceIdType.MESH,
                ).start()

        ag_ref[pl.ds(my_pos, 1)] = acc_ref[...].reshape(1, CH, D_MODEL)
        for src in range(N_DEV):
            @pl.when(src != my_pos)
            def _():
                pltpu.make_async_remote_copy(
                    src_ref=acc_ref,
                    dst_ref=ag_ref.at[src],
                    send_sem=send2.at[src],
                    recv_sem=recv2.at[src],
                    device_id=(src,),
                    device_id_type=pl.DeviceIdType.MESH,
                ).wait_recv()

        out_ref[...] = ag_ref[...].reshape(ROWS, D_MODEL).astype(jnp.float32)

        for dst in range(N_DEV):
            @pl.when(dst != my_pos)
            def _():
                pltpu.make_async_remote_copy(
                    src_ref=partial_ref.at[pl.ds(dst * CH, CH), :],
                    dst_ref=rs_ref.at[my_pos],
                    send_sem=send1.at[dst],
                    recv_sem=recv1.at[my_pos],
                    device_id=(dst,),
                    device_id_type=pl.DeviceIdType.MESH,
                ).wait_send()
                pltpu.make_async_remote_copy(
                    src_ref=acc_ref,
                    dst_ref=ag_ref.at[my_pos],
                    send_sem=send2.at[dst],
                    recv_sem=recv2.at[my_pos],
                    device_id=(dst,),
                    device_id_type=pl.DeviceIdType.MESH,
                ).wait_send()

    out = pl.pallas_call(
        body,
        out_shape=jax.ShapeDtypeStruct((ROWS, D_MODEL), jnp.float32),
        in_specs=[pl.BlockSpec(memory_space=pltpu.VMEM)] * 5,
        out_specs=pl.BlockSpec(memory_space=pltpu.VMEM),
        scratch_shapes=[
            pltpu.VMEM((ROWS, HL * DH), jnp.float32),
            pltpu.VMEM((ROWS, D_MODEL), jnp.bfloat16),
            pltpu.VMEM((N_DEV, CH, D_MODEL), jnp.bfloat16),
            pltpu.VMEM((CH, D_MODEL), jnp.bfloat16),
            pltpu.VMEM((N_DEV, CH, D_MODEL), jnp.bfloat16),
            pltpu.SemaphoreType.DMA((N_DEV,)),
            pltpu.SemaphoreType.DMA((N_DEV,)),
            pltpu.SemaphoreType.DMA((N_DEV,)),
            pltpu.SemaphoreType.DMA((N_DEV,)),
        ],
        compiler_params=pltpu.CompilerParams(collective_id=0),
    )(x2d, Wq, K_loc, V_loc, Wo)
    return out.reshape(B, SQ, D_MODEL)
